# baseline (speedup 1.0000x reference)
import numpy as np
import jax
import jax.numpy as jnp
from jax import lax
from jax.experimental import pallas as pl
from jax.experimental.pallas import tpu as pltpu
from jax.experimental.pallas import tpu_sc as plsc
from jax._src import config as _jax_src_config

N_LEVELS = 12
N_FEATURES = 2
HASHMAP_SIZE = 2 ** 19
MASK = np.int32(HASHMAP_SIZE - 1)
BASE_RES = 16
GROWTH = 1.38
RES = [int(np.floor(BASE_RES * GROWTH ** l)) for l in range(N_LEVELS)]
P1 = np.uint32(2654435761).astype(np.int32)
P2 = np.int32(805459861)
N_PTS = 524288
N_OUT = N_LEVELS * N_FEATURES

NW = 32
PTS_PER_W = N_PTS // NW
CHUNK = 512
GROUPS = CHUNK // 16
N_CHUNKS = PTS_PER_W // CHUNK
HALF = 8 * CHUNK
N_IDX = 2 * HALF


def _fori32(n, body):
    lax.fori_loop(0, n, lambda i, c: (body(i), c)[1], None, unroll=False)


def _body(x_hbm, tab_hbm, out_hbm, x_v, idx_v, rows_v, out_v, sem):
    wid = lax.axis_index("s") * np.int32(2) + lax.axis_index("c")
    base = wid * np.int32(PTS_PER_W)
    iota = lax.iota(jnp.int32, 16)

    def chunk_body(ch):
        cbase = base + ch * np.int32(CHUNK)
        pltpu.sync_copy(x_hbm.at[pl.ds(cbase * np.int32(3), 3 * CHUNK)], x_v)
        for l in range(N_LEVELS):
            res = np.float32(RES[l])
            lvl_off = np.int32(l * HASHMAP_SIZE * N_FEATURES)

            def hash_body(g):
                off = g * np.int32(16)
                p3 = (off + iota) * np.int32(3)
                xi = plsc.load_gather(x_v, [p3])
                yi = plsc.load_gather(x_v, [p3 + np.int32(1)])
                zi = plsc.load_gather(x_v, [p3 + np.int32(2)])
                fx = (xi * res).astype(jnp.int32)
                fy = (yi * res).astype(jnp.int32)
                fz = (zi * res).astype(jnp.int32)
                hy0 = fy * P1
                hz0 = fz * P2
                hxy = (fx ^ hy0, (fx + np.int32(1)) ^ hy0, fx ^ (hy0 + P1),
                       (fx + np.int32(1)) ^ (hy0 + P1))
                for c in range(8):
                    hz = (hz0 + P2) if (c & 4) else hz0
                    h = (hxy[c & 3] ^ hz) & MASK
                    w0 = lax.shift_left(h, np.int32(1)) + lvl_off
                    pos = np.int32(c * CHUNK) + off
                    idx_v[pl.ds(pos, 16)] = w0
                    idx_v[pl.ds(np.int32(HALF + c * CHUNK) + off, 16)] = (
                        w0 + np.int32(1))

            _fori32(GROUPS, hash_body)
            pltpu.async_copy(tab_hbm.at[idx_v], rows_v, sem).wait()

            def comb_body(g):
                off = g * np.int32(16)
                p3 = (off + iota) * np.int32(3)
                xi = plsc.load_gather(x_v, [p3])
                yi = plsc.load_gather(x_v, [p3 + np.int32(1)])
                zi = plsc.load_gather(x_v, [p3 + np.int32(2)])
                xs = xi * res
                ys = yi * res
                zs = zi * res
                wx = xs - xs.astype(jnp.int32).astype(jnp.float32)
                wy = ys - ys.astype(jnp.int32).astype(jnp.float32)
                wz = zs - zs.astype(jnp.int32).astype(jnp.float32)
                one = np.float32(1.0)
                ax = (one - wx, wx)
                ay = (one - wy, wy)
                az = (one - wz, wz)
                wxy = (ax[0] * ay[0], ax[1] * ay[0], ax[0] * ay[1],
                       ax[1] * ay[1])
                acc0 = jnp.zeros((16,), jnp.float32)
                acc1 = jnp.zeros((16,), jnp.float32)
                for c in range(8):
                    wc = wxy[c & 3] * az[(c >> 2) & 1]
                    pos = np.int32(c * CHUNK) + off
                    f0 = rows_v[pl.ds(pos, 16)]
                    f1 = rows_v[pl.ds(np.int32(HALF + c * CHUNK) + off, 16)]
                    acc0 = acc0 + wc * f0
                    acc1 = acc1 + wc * f1
                rows24 = (off + iota) * np.int32(N_OUT)
                plsc.store_scatter(out_v, [rows24 + np.int32(2 * l)], acc0)
                plsc.store_scatter(out_v, [rows24 + np.int32(2 * l + 1)],
                                   acc1)

            _fori32(GROUPS, comb_body)
        pltpu.sync_copy(out_v,
                        out_hbm.at[pl.ds(cbase * np.int32(N_OUT),
                                         CHUNK * N_OUT)])

    _fori32(N_CHUNKS, chunk_body)


@jax.jit
def _hash_grid(x_flat, tab_flat):
    mesh = plsc.VectorSubcoreMesh(core_axis_name="c", subcore_axis_name="s")
    out = pl.kernel(
        _body,
        out_type=jax.ShapeDtypeStruct((N_PTS * N_OUT,), jnp.float32),
        mesh=mesh,
        compiler_params=pltpu.CompilerParams(needs_layout_passes=False,
                                             use_tc_tiling_on_sc=False),
        scratch_types=[
            pltpu.VMEM((3 * CHUNK,), jnp.float32),
            pltpu.VMEM((N_IDX,), jnp.int32),
            pltpu.VMEM((N_IDX,), jnp.float32),
            pltpu.VMEM((CHUNK * N_OUT,), jnp.float32),
            pltpu.SemaphoreType.DMA,
        ],
    )(x_flat, tab_flat)
    return out.reshape(N_PTS, N_OUT)


def kernel(x, tables):
    x_flat = x.astype(jnp.float32).reshape(-1)
    tab_flat = tables.astype(jnp.float32).reshape(-1)
    with _jax_src_config.enable_x64(False):
        return _hash_grid(x_flat, tab_flat)


# speedup vs baseline: 2.4338x; 2.4338x over previous
import numpy as np
import jax
import jax.numpy as jnp
from jax import lax
from jax.experimental import pallas as pl
from jax.experimental.pallas import tpu as pltpu
from jax.experimental.pallas import tpu_sc as plsc
from jax._src import config as _jax_src_config

N_LEVELS = 12
N_FEATURES = 2
HASHMAP_SIZE = 2 ** 19
MASK = np.int32(HASHMAP_SIZE - 1)
BASE_RES = 16
GROWTH = 1.38
RES = [int(np.floor(BASE_RES * GROWTH ** l)) for l in range(N_LEVELS)]
P1 = np.uint32(2654435761).astype(np.int32)
P2 = np.int32(805459861)
N_PTS = 524288
N_OUT = N_LEVELS * N_FEATURES

NW = 32
PTS_PER_W = N_PTS // NW
CHUNK = 512
GROUPS = CHUNK // 16
N_CHUNKS = PTS_PER_W // CHUNK
HALF = 8 * CHUNK
N_IDX = 2 * HALF


def _fori32(n, body):
    lax.fori_loop(0, n, lambda i, c: (body(i), c)[1], None, unroll=False)


def _body(x_hbm, tab_hbm, out_hbm, x_v, idx_v, rows_v, out_v, sem):
    wid = lax.axis_index("s") * np.int32(2) + lax.axis_index("c")
    base = wid * np.int32(PTS_PER_W)
    iota = lax.iota(jnp.int32, 16)

    def chunk_body(ch):
        cbase = base + ch * np.int32(CHUNK)
        pltpu.sync_copy(x_hbm.at[pl.ds(cbase * np.int32(3), 3 * CHUNK)], x_v)
        for l in range(N_LEVELS):
            res = np.float32(RES[l])
            lvl_off = np.int32(l * HASHMAP_SIZE * N_FEATURES)

            def hash_body(g):
                off = g * np.int32(16)
                p3 = (off + iota) * np.int32(3)
                xi = plsc.load_gather(x_v, [p3])
                yi = plsc.load_gather(x_v, [p3 + np.int32(1)])
                zi = plsc.load_gather(x_v, [p3 + np.int32(2)])
                fx = (xi * res).astype(jnp.int32)
                fy = (yi * res).astype(jnp.int32)
                fz = (zi * res).astype(jnp.int32)
                hy0 = fy * P1
                hz0 = fz * P2
                hxy = (fx ^ hy0, (fx + np.int32(1)) ^ hy0, fx ^ (hy0 + P1),
                       (fx + np.int32(1)) ^ (hy0 + P1))
                for c in range(8):
                    hz = (hz0 + P2) if (c & 4) else hz0
                    h = (hxy[c & 3] ^ hz) & MASK
                    w0 = h + (h & np.int32(-128)) + lvl_off
                    pos = np.int32(c * CHUNK) + off
                    idx_v[pl.ds(pos, 16)] = w0
                    idx_v[pl.ds(np.int32(HALF + c * CHUNK) + off, 16)] = (
                        w0 + np.int32(128))

            _fori32(GROUPS, hash_body)
            pltpu.async_copy(tab_hbm.at[idx_v], rows_v, sem).wait()

            def comb_body(g):
                off = g * np.int32(16)
                p3 = (off + iota) * np.int32(3)
                xi = plsc.load_gather(x_v, [p3])
                yi = plsc.load_gather(x_v, [p3 + np.int32(1)])
                zi = plsc.load_gather(x_v, [p3 + np.int32(2)])
                xs = xi * res
                ys = yi * res
                zs = zi * res
                wx = xs - xs.astype(jnp.int32).astype(jnp.float32)
                wy = ys - ys.astype(jnp.int32).astype(jnp.float32)
                wz = zs - zs.astype(jnp.int32).astype(jnp.float32)
                one = np.float32(1.0)
                ax = (one - wx, wx)
                ay = (one - wy, wy)
                az = (one - wz, wz)
                wxy = (ax[0] * ay[0], ax[1] * ay[0], ax[0] * ay[1],
                       ax[1] * ay[1])
                acc0 = jnp.zeros((16,), jnp.float32)
                acc1 = jnp.zeros((16,), jnp.float32)
                for c in range(8):
                    wc = wxy[c & 3] * az[(c >> 2) & 1]
                    pos = np.int32(c * CHUNK) + off
                    f0 = rows_v[pl.ds(pos, 16)]
                    f1 = rows_v[pl.ds(np.int32(HALF + c * CHUNK) + off, 16)]
                    acc0 = acc0 + wc * f0
                    acc1 = acc1 + wc * f1
                rows24 = (off + iota) * np.int32(N_OUT)
                plsc.store_scatter(out_v, [rows24 + np.int32(2 * l)], acc0)
                plsc.store_scatter(out_v, [rows24 + np.int32(2 * l + 1)],
                                   acc1)

            _fori32(GROUPS, comb_body)
        pltpu.sync_copy(out_v,
                        out_hbm.at[pl.ds(cbase * np.int32(N_OUT),
                                         CHUNK * N_OUT)])

    _fori32(N_CHUNKS, chunk_body)


@jax.jit
def _hash_grid(x_flat, tab_flat):
    mesh = plsc.VectorSubcoreMesh(core_axis_name="c", subcore_axis_name="s")
    out = pl.kernel(
        _body,
        out_type=jax.ShapeDtypeStruct((N_PTS * N_OUT,), jnp.float32),
        mesh=mesh,
        compiler_params=pltpu.CompilerParams(needs_layout_passes=False,
                                             use_tc_tiling_on_sc=False),
        scratch_types=[
            pltpu.VMEM((3 * CHUNK,), jnp.float32),
            pltpu.VMEM((N_IDX,), jnp.int32),
            pltpu.VMEM((N_IDX,), jnp.float32),
            pltpu.VMEM((CHUNK * N_OUT,), jnp.float32),
            pltpu.SemaphoreType.DMA,
        ],
    )(x_flat, tab_flat)
    return out.reshape(N_PTS, N_OUT)


def kernel(x, tables):
    x_flat = x.astype(jnp.float32).reshape(-1)
    tab_flat = tables.astype(jnp.float32).reshape(
        N_LEVELS, HASHMAP_SIZE // 128, 128, N_FEATURES).transpose(
        0, 1, 3, 2).reshape(-1)
    with _jax_src_config.enable_x64(False):
        return _hash_grid(x_flat, tab_flat)


# speedup vs baseline: 3.4699x; 1.4257x over previous
import numpy as np
import jax
import jax.numpy as jnp
from jax import lax
from jax.experimental import pallas as pl
from jax.experimental.pallas import tpu as pltpu
from jax.experimental.pallas import tpu_sc as plsc
from jax._src import config as _jax_src_config

N_LEVELS = 12
N_FEATURES = 2
HASHMAP_SIZE = 2 ** 19
MASK = np.int32(HASHMAP_SIZE - 1)
BASE_RES = 16
GROWTH = 1.38
RES = [int(np.floor(BASE_RES * GROWTH ** l)) for l in range(N_LEVELS)]
P1 = np.uint32(2654435761).astype(np.int32)
P2 = np.int32(805459861)
N_PTS = 524288
N_OUT = N_LEVELS * N_FEATURES
N_WORDS = N_LEVELS * HASHMAP_SIZE * N_FEATURES

ROWS_PER_SUPER = 8
SUPER_W = N_FEATURES * ROWS_PER_SUPER
N_SUPER = N_WORDS // SUPER_W
SUPER_PER_LVL = HASHMAP_SIZE // ROWS_PER_SUPER

NW = 32
PTS_PER_W = N_PTS // NW
CHUNK = 512
GROUPS = CHUNK // 16
N_CHUNKS = PTS_PER_W // CHUNK
N_IDX = 8 * CHUNK

N_BLOCKS = N_WORDS // 256
BLK_PER_W = N_BLOCKS // NW
BLK_STEP = 16
N_STEPS = BLK_PER_W // BLK_STEP


def _fori32(n, body):
    lax.fori_loop(0, n, lambda i, c: (body(i), c)[1], None, unroll=False)


def _wid():
    return lax.axis_index("s") * np.int32(2) + lax.axis_index("c")


def _il_body(tab_hbm, out_hbm, in_v, out_v):
    wid = _wid()
    iota = lax.iota(jnp.int32, 16)
    blk0 = wid * np.int32(BLK_PER_W)

    def step_body(s):
        b0 = blk0 + s * np.int32(BLK_STEP)
        woff = b0 * np.int32(256)
        pltpu.sync_copy(tab_hbm.at[pl.ds(woff, BLK_STEP * 256)], in_v)

        def blk_body(b):
            ib = b * np.int32(256)
            rb = lax.shift_left(b, np.int32(4))
            for k in range(8):
                a = in_v[pl.ds(ib + np.int32(16 * k), 16)]
                bv = in_v[pl.ds(ib + np.int32(128 + 16 * k), 16)]
                pos0 = np.int32(32 * k) + lax.shift_left(iota, np.int32(1))
                for f, vec in ((0, a), (1, bv)):
                    pos = pos0 + np.int32(f)
                    row = rb + lax.shift_right_logical(pos, np.int32(4))
                    col = pos & np.int32(15)
                    plsc.store_scatter(out_v, [row, col], vec)

        _fori32(BLK_STEP, blk_body)
        pltpu.sync_copy(out_v,
                        out_hbm.at[pl.ds(b0 * np.int32(16), BLK_STEP * 16)])

    _fori32(N_STEPS, step_body)


def _body(x_hbm, tab_hbm, out_hbm, x_v, idx_v, m_v, rows_v, out_v, sem):
    wid = _wid()
    base = wid * np.int32(PTS_PER_W)
    iota = lax.iota(jnp.int32, 16)

    def chunk_body(ch):
        cbase = base + ch * np.int32(CHUNK)
        pltpu.sync_copy(x_hbm.at[pl.ds(cbase * np.int32(3), 3 * CHUNK)], x_v)
        for l in range(N_LEVELS):
            res = np.float32(RES[l])
            lvl_off = np.int32(l * SUPER_PER_LVL)

            def hash_body(g):
                off = g * np.int32(16)
                p3 = (off + iota) * np.int32(3)
                xi = plsc.load_gather(x_v, [p3])
                yi = plsc.load_gather(x_v, [p3 + np.int32(1)])
                zi = plsc.load_gather(x_v, [p3 + np.int32(2)])
                fx = (xi * res).astype(jnp.int32)
                fy = (yi * res).astype(jnp.int32)
                fz = (zi * res).astype(jnp.int32)
                hy0 = fy * P1
                hz0 = fz * P2
                hxy = (fx ^ hy0, (fx + np.int32(1)) ^ hy0, fx ^ (hy0 + P1),
                       (fx + np.int32(1)) ^ (hy0 + P1))
                for c in range(8):
                    hz = (hz0 + P2) if (c & 4) else hz0
                    h = (hxy[c & 3] ^ hz) & MASK
                    pos = np.int32(c * CHUNK) + off
                    idx_v[pl.ds(pos, 16)] = (
                        lax.shift_right_logical(h, np.int32(3)) + lvl_off)
                    m_v[pl.ds(pos, 16)] = lax.shift_left(
                        h & np.int32(7), np.int32(1))

            _fori32(GROUPS, hash_body)
            pltpu.async_copy(tab_hbm.at[idx_v], rows_v, sem).wait()

            def comb_body(g):
                off = g * np.int32(16)
                p3 = (off + iota) * np.int32(3)
                xi = plsc.load_gather(x_v, [p3])
                yi = plsc.load_gather(x_v, [p3 + np.int32(1)])
                zi = plsc.load_gather(x_v, [p3 + np.int32(2)])
                xs = xi * res
                ys = yi * res
                zs = zi * res
                wx = xs - xs.astype(jnp.int32).astype(jnp.float32)
                wy = ys - ys.astype(jnp.int32).astype(jnp.float32)
                wz = zs - zs.astype(jnp.int32).astype(jnp.float32)
                one = np.float32(1.0)
                ax = (one - wx, wx)
                ay = (one - wy, wy)
                az = (one - wz, wz)
                wxy = (ax[0] * ay[0], ax[1] * ay[0], ax[0] * ay[1],
                       ax[1] * ay[1])
                acc0 = jnp.zeros((16,), jnp.float32)
                acc1 = jnp.zeros((16,), jnp.float32)
                for c in range(8):
                    wc = wxy[c & 3] * az[(c >> 2) & 1]
                    pos = np.int32(c * CHUNK) + off
                    m0 = m_v[pl.ds(pos, 16)]
                    srow = pos + iota
                    f0 = plsc.load_gather(rows_v, [srow, m0])
                    f1 = plsc.load_gather(rows_v, [srow, m0 + np.int32(1)])
                    acc0 = acc0 + wc * f0
                    acc1 = acc1 + wc * f1
                rows24 = (off + iota) * np.int32(N_OUT)
                plsc.store_scatter(out_v, [rows24 + np.int32(2 * l)], acc0)
                plsc.store_scatter(out_v, [rows24 + np.int32(2 * l + 1)],
                                   acc1)

            _fori32(GROUPS, comb_body)
        pltpu.sync_copy(out_v,
                        out_hbm.at[pl.ds(cbase * np.int32(N_OUT),
                                         CHUNK * N_OUT)])

    _fori32(N_CHUNKS, chunk_body)


@jax.jit
def _hash_grid(x_flat, tab_flat):
    mesh = plsc.VectorSubcoreMesh(core_axis_name="c", subcore_axis_name="s")
    params = pltpu.CompilerParams(needs_layout_passes=False,
                                  use_tc_tiling_on_sc=False)
    tab_il = pl.kernel(
        _il_body,
        out_type=jax.ShapeDtypeStruct((N_SUPER, SUPER_W), jnp.float32),
        mesh=mesh,
        compiler_params=params,
        scratch_types=[
            pltpu.VMEM((BLK_STEP * 256,), jnp.float32),
            pltpu.VMEM((BLK_STEP * 16, SUPER_W), jnp.float32),
        ],
    )(tab_flat)
    out = pl.kernel(
        _body,
        out_type=jax.ShapeDtypeStruct((N_PTS * N_OUT,), jnp.float32),
        mesh=mesh,
        compiler_params=params,
        scratch_types=[
            pltpu.VMEM((3 * CHUNK,), jnp.float32),
            pltpu.VMEM((N_IDX,), jnp.int32),
            pltpu.VMEM((N_IDX,), jnp.int32),
            pltpu.VMEM((N_IDX, SUPER_W), jnp.float32),
            pltpu.VMEM((CHUNK * N_OUT,), jnp.float32),
            pltpu.SemaphoreType.DMA,
        ],
    )(x_flat, tab_il)
    return out.reshape(N_PTS, N_OUT)


def kernel(x, tables):
    x_flat = x.astype(jnp.float32).reshape(-1)
    tab_flat = tables.astype(jnp.float32).reshape(
        N_LEVELS, HASHMAP_SIZE // 128, 128, N_FEATURES).transpose(
        0, 1, 3, 2).reshape(-1)
    with _jax_src_config.enable_x64(False):
        return _hash_grid(x_flat, tab_flat)


# speedup vs baseline: 4.5186x; 1.3023x over previous
import numpy as np
import jax
import jax.numpy as jnp
from jax import lax
from jax.experimental import pallas as pl
from jax.experimental.pallas import tpu as pltpu
from jax.experimental.pallas import tpu_sc as plsc
from jax._src import config as _jax_src_config

N_LEVELS = 12
N_FEATURES = 2
HASHMAP_SIZE = 2 ** 19
MASK = np.int32(HASHMAP_SIZE - 1)
BASE_RES = 16
GROWTH = 1.38
RES = [int(np.floor(BASE_RES * GROWTH ** l)) for l in range(N_LEVELS)]
P1 = np.uint32(2654435761).astype(np.int32)
P2 = np.int32(805459861)
N_PTS = 524288
N_OUT = N_LEVELS * N_FEATURES
N_WORDS = N_LEVELS * HASHMAP_SIZE * N_FEATURES

ROWS_PER_SUPER = 8
SUPER_W = N_FEATURES * ROWS_PER_SUPER
N_SUPER = N_WORDS // SUPER_W
SUPER_PER_LVL = HASHMAP_SIZE // ROWS_PER_SUPER

NW = 32
PTS_PER_W = N_PTS // NW
CHUNK = 256
GROUPS = CHUNK // 16
N_CHUNKS = PTS_PER_W // CHUNK
N_IDX = 8 * CHUNK

N_BLOCKS = N_WORDS // 256
BLK_PER_W = N_BLOCKS // NW
BLK_STEP = 16
N_STEPS = BLK_PER_W // BLK_STEP


def _fori32(n, body):
    lax.fori_loop(0, n, lambda i, c: (body(i), c)[1], None, unroll=False)


def _wid():
    return lax.axis_index("s") * np.int32(2) + lax.axis_index("c")


def _il_body(tab_hbm, out_hbm, in_v, out_v):
    wid = _wid()
    iota = lax.iota(jnp.int32, 16)
    blk0 = wid * np.int32(BLK_PER_W)

    def step_body(s):
        b0 = blk0 + s * np.int32(BLK_STEP)
        woff = b0 * np.int32(256)
        pltpu.sync_copy(tab_hbm.at[pl.ds(woff, BLK_STEP * 256)], in_v)

        def blk_body(b):
            ib = b * np.int32(256)
            rb = lax.shift_left(b, np.int32(4))
            for k in range(8):
                a = in_v[pl.ds(ib + np.int32(16 * k), 16)]
                bv = in_v[pl.ds(ib + np.int32(128 + 16 * k), 16)]
                pos0 = np.int32(32 * k) + lax.shift_left(iota, np.int32(1))
                for f, vec in ((0, a), (1, bv)):
                    pos = pos0 + np.int32(f)
                    row = rb + lax.shift_right_logical(pos, np.int32(4))
                    col = pos & np.int32(15)
                    plsc.store_scatter(out_v, [row, col], vec)

        _fori32(BLK_STEP, blk_body)
        pltpu.sync_copy(out_v,
                        out_hbm.at[pl.ds(b0 * np.int32(16), BLK_STEP * 16)])

    _fori32(N_STEPS, step_body)


def _body(x_hbm, tab_hbm, out_hbm, x_v, idx_a, m_a, rows_a,
          idx_b, m_b, rows_b, out_v, sem_a, sem_b):
    wid = _wid()
    base = wid * np.int32(PTS_PER_W)
    iota = lax.iota(jnp.int32, 16)

    def chunk_body(ch):
        cbase = base + ch * np.int32(CHUNK)
        pltpu.sync_copy(x_hbm.at[pl.ds(cbase * np.int32(3), 3 * CHUNK)], x_v)

        def make_hash(l, idx_v, m_v):
            res = np.float32(RES[l])
            lvl_off = np.int32(l * SUPER_PER_LVL)

            def hash_body(g):
                off = g * np.int32(16)
                p3 = (off + iota) * np.int32(3)
                xi = plsc.load_gather(x_v, [p3])
                yi = plsc.load_gather(x_v, [p3 + np.int32(1)])
                zi = plsc.load_gather(x_v, [p3 + np.int32(2)])
                fx = (xi * res).astype(jnp.int32)
                fy = (yi * res).astype(jnp.int32)
                fz = (zi * res).astype(jnp.int32)
                hy0 = fy * P1
                hz0 = fz * P2
                hxy = (fx ^ hy0, (fx + np.int32(1)) ^ hy0, fx ^ (hy0 + P1),
                       (fx + np.int32(1)) ^ (hy0 + P1))
                for c in range(8):
                    hz = (hz0 + P2) if (c & 4) else hz0
                    h = (hxy[c & 3] ^ hz) & MASK
                    pos = np.int32(c * CHUNK) + off
                    idx_v[pl.ds(pos, 16)] = (
                        lax.shift_right_logical(h, np.int32(3)) + lvl_off)
                    m_v[pl.ds(pos, 16)] = lax.shift_left(
                        h & np.int32(7), np.int32(1))

            _fori32(GROUPS, hash_body)

        def make_comb(l, m_v, rows_v):
            res = np.float32(RES[l])

            def comb_body(g):
                off = g * np.int32(16)
                p3 = (off + iota) * np.int32(3)
                xi = plsc.load_gather(x_v, [p3])
                yi = plsc.load_gather(x_v, [p3 + np.int32(1)])
                zi = plsc.load_gather(x_v, [p3 + np.int32(2)])
                xs = xi * res
                ys = yi * res
                zs = zi * res
                wx = xs - xs.astype(jnp.int32).astype(jnp.float32)
                wy = ys - ys.astype(jnp.int32).astype(jnp.float32)
                wz = zs - zs.astype(jnp.int32).astype(jnp.float32)
                one = np.float32(1.0)
                ax = (one - wx, wx)
                ay = (one - wy, wy)
                az = (one - wz, wz)
                wxy = (ax[0] * ay[0], ax[1] * ay[0], ax[0] * ay[1],
                       ax[1] * ay[1])
                acc0 = jnp.zeros((16,), jnp.float32)
                acc1 = jnp.zeros((16,), jnp.float32)
                for c in range(8):
                    wc = wxy[c & 3] * az[(c >> 2) & 1]
                    pos = np.int32(c * CHUNK) + off
                    m0 = m_v[pl.ds(pos, 16)]
                    srow = pos + iota
                    f0 = plsc.load_gather(rows_v, [srow, m0])
                    f1 = plsc.load_gather(rows_v, [srow, m0 + np.int32(1)])
                    acc0 = acc0 + wc * f0
                    acc1 = acc1 + wc * f1
                rows24 = (off + iota) * np.int32(N_OUT)
                plsc.store_scatter(out_v, [rows24 + np.int32(2 * l)], acc0)
                plsc.store_scatter(out_v, [rows24 + np.int32(2 * l + 1)],
                                   acc1)

            _fori32(GROUPS, comb_body)

        bufs = ((idx_a, m_a, rows_a, sem_a), (idx_b, m_b, rows_b, sem_b))
        make_hash(0, idx_a, m_a)
        pltpu.async_copy(tab_hbm.at[idx_a], rows_a, sem_a)
        for l in range(N_LEVELS):
            idx_c, m_c, rows_c, sem_c = bufs[l & 1]
            if l + 1 < N_LEVELS:
                idx_n, m_n, rows_n, sem_n = bufs[(l + 1) & 1]
                make_hash(l + 1, idx_n, m_n)
                pltpu.async_copy(tab_hbm.at[idx_n], rows_n, sem_n)
            pltpu.make_async_copy(tab_hbm.at[idx_c], rows_c, sem_c).wait()
            make_comb(l, m_c, rows_c)
        pltpu.sync_copy(out_v,
                        out_hbm.at[pl.ds(cbase * np.int32(N_OUT),
                                         CHUNK * N_OUT)])

    _fori32(N_CHUNKS, chunk_body)


@jax.jit
def _hash_grid(x_flat, tab_flat):
    mesh = plsc.VectorSubcoreMesh(core_axis_name="c", subcore_axis_name="s")
    params = pltpu.CompilerParams(needs_layout_passes=False,
                                  use_tc_tiling_on_sc=False)
    tab_il = pl.kernel(
        _il_body,
        out_type=jax.ShapeDtypeStruct((N_SUPER, SUPER_W), jnp.float32),
        mesh=mesh,
        compiler_params=params,
        scratch_types=[
            pltpu.VMEM((BLK_STEP * 256,), jnp.float32),
            pltpu.VMEM((BLK_STEP * 16, SUPER_W), jnp.float32),
        ],
    )(tab_flat)
    out = pl.kernel(
        _body,
        out_type=jax.ShapeDtypeStruct((N_PTS * N_OUT,), jnp.float32),
        mesh=mesh,
        compiler_params=params,
        scratch_types=[
            pltpu.VMEM((3 * CHUNK,), jnp.float32),
            pltpu.VMEM((N_IDX,), jnp.int32),
            pltpu.VMEM((N_IDX,), jnp.int32),
            pltpu.VMEM((N_IDX, SUPER_W), jnp.float32),
            pltpu.VMEM((N_IDX,), jnp.int32),
            pltpu.VMEM((N_IDX,), jnp.int32),
            pltpu.VMEM((N_IDX, SUPER_W), jnp.float32),
            pltpu.VMEM((CHUNK * N_OUT,), jnp.float32),
            pltpu.SemaphoreType.DMA,
            pltpu.SemaphoreType.DMA,
        ],
    )(x_flat, tab_il)
    return out.reshape(N_PTS, N_OUT)


def kernel(x, tables):
    x_flat = x.astype(jnp.float32).reshape(-1)
    tab_flat = tables.astype(jnp.float32).reshape(
        N_LEVELS, HASHMAP_SIZE // 128, 128, N_FEATURES).transpose(
        0, 1, 3, 2).reshape(-1)
    with _jax_src_config.enable_x64(False):
        return _hash_grid(x_flat, tab_flat)


# speedup vs baseline: 4.9811x; 1.1023x over previous
import numpy as np
import jax
import jax.numpy as jnp
from jax import lax
from jax.experimental import pallas as pl
from jax.experimental.pallas import tpu as pltpu
from jax.experimental.pallas import tpu_sc as plsc
from jax._src import config as _jax_src_config

N_LEVELS = 12
N_FEATURES = 2
HASHMAP_SIZE = 2 ** 19
MASK = np.int32(HASHMAP_SIZE - 1)
BASE_RES = 16
GROWTH = 1.38
RES = [int(np.floor(BASE_RES * GROWTH ** l)) for l in range(N_LEVELS)]
P1 = np.uint32(2654435761).astype(np.int32)
P2 = np.int32(805459861)
N_PTS = 524288
N_OUT = N_LEVELS * N_FEATURES
N_WORDS = N_LEVELS * HASHMAP_SIZE * N_FEATURES

ROWS_PER_SUPER = 8
SUPER_W = N_FEATURES * ROWS_PER_SUPER
N_SUPER = N_WORDS // SUPER_W
SUPER_PER_LVL = HASHMAP_SIZE // ROWS_PER_SUPER

NW = 32
PTS_PER_W = N_PTS // NW
CHUNK = 256
GROUPS = CHUNK // 16
N_CHUNKS = PTS_PER_W // CHUNK
N_IDX = 8 * CHUNK

N_BLOCKS = N_WORDS // 256
BLK_PER_W = N_BLOCKS // NW
BLK_STEP = 16
N_STEPS = BLK_PER_W // BLK_STEP


def _fori32(n, body):
    lax.fori_loop(0, n, lambda i, c: (body(i), c)[1], None, unroll=False)


def _wid():
    return lax.axis_index("s") * np.int32(2) + lax.axis_index("c")


def _il_body(tab_hbm, out_hbm, in_v, out_v):
    wid = _wid()
    iota = lax.iota(jnp.int32, 16)
    blk0 = wid * np.int32(BLK_PER_W)

    def step_body(s):
        b0 = blk0 + s * np.int32(BLK_STEP)
        woff = b0 * np.int32(256)
        pltpu.sync_copy(tab_hbm.at[pl.ds(woff, BLK_STEP * 256)], in_v)

        def blk_body(b):
            ib = b * np.int32(256)
            rb = lax.shift_left(b, np.int32(4))
            for k in range(8):
                a = in_v[pl.ds(ib + np.int32(16 * k), 16)]
                bv = in_v[pl.ds(ib + np.int32(128 + 16 * k), 16)]
                pos0 = np.int32(32 * k) + lax.shift_left(iota, np.int32(1))
                for f, vec in ((0, a), (1, bv)):
                    pos = pos0 + np.int32(f)
                    row = rb + lax.shift_right_logical(pos, np.int32(4))
                    col = pos & np.int32(15)
                    plsc.store_scatter(out_v, [row, col], vec)

        _fori32(BLK_STEP, blk_body)
        pltpu.sync_copy(out_v,
                        out_hbm.at[pl.ds(b0 * np.int32(16), BLK_STEP * 16)])

    _fori32(N_STEPS, step_body)


def _body(x_hbm, tab_hbm, out_hbm, x_v, idx_a, m_a, rows_a,
          idx_b, m_b, rows_b, out_v, sem_a, sem_b):
    wid = _wid()
    base = wid * np.int32(PTS_PER_W)
    iota = lax.iota(jnp.int32, 16)

    def chunk_body(ch):
        cbase = base + ch * np.int32(CHUNK)
        pltpu.sync_copy(x_hbm.at[pl.ds(cbase * np.int32(3), 3 * CHUNK)], x_v)

        def make_hash(l, idx_v, m_v):
            res = np.float32(RES[l])
            lvl_off = np.int32(l * SUPER_PER_LVL)

            def hash_body(g):
                off = g * np.int32(16)
                p3 = (off + iota) * np.int32(3)
                xi = plsc.load_gather(x_v, [p3])
                yi = plsc.load_gather(x_v, [p3 + np.int32(1)])
                zi = plsc.load_gather(x_v, [p3 + np.int32(2)])
                fx = (xi * res).astype(jnp.int32)
                fy = (yi * res).astype(jnp.int32)
                fz = (zi * res).astype(jnp.int32)
                hy0 = fy * P1
                hz0 = fz * P2
                hxy = (fx ^ hy0, (fx + np.int32(1)) ^ hy0, fx ^ (hy0 + P1),
                       (fx + np.int32(1)) ^ (hy0 + P1))
                for c in range(8):
                    hz = (hz0 + P2) if (c & 4) else hz0
                    h = (hxy[c & 3] ^ hz) & MASK
                    pos = np.int32(c * CHUNK) + off
                    idx_v[pl.ds(pos, 16)] = (
                        lax.shift_right_logical(h, np.int32(3)) + lvl_off)
                    m_v[pl.ds(pos, 16)] = lax.shift_left(
                        h & np.int32(7), np.int32(1))

            _fori32(GROUPS, hash_body)

        def make_comb(l, m_v, rows_v):
            res = np.float32(RES[l])

            def comb_body(g):
                off = g * np.int32(16)
                p3 = (off + iota) * np.int32(3)
                xi = plsc.load_gather(x_v, [p3])
                yi = plsc.load_gather(x_v, [p3 + np.int32(1)])
                zi = plsc.load_gather(x_v, [p3 + np.int32(2)])
                xs = xi * res
                ys = yi * res
                zs = zi * res
                wx = xs - xs.astype(jnp.int32).astype(jnp.float32)
                wy = ys - ys.astype(jnp.int32).astype(jnp.float32)
                wz = zs - zs.astype(jnp.int32).astype(jnp.float32)
                one = np.float32(1.0)
                ax = (one - wx, wx)
                ay = (one - wy, wy)
                az = (one - wz, wz)
                wxy = (ax[0] * ay[0], ax[1] * ay[0], ax[0] * ay[1],
                       ax[1] * ay[1])
                acc0 = jnp.zeros((16,), jnp.float32)
                acc1 = jnp.zeros((16,), jnp.float32)
                for c in range(8):
                    wc = wxy[c & 3] * az[(c >> 2) & 1]
                    pos = np.int32(c * CHUNK) + off
                    m0 = m_v[pl.ds(pos, 16)]
                    srow = pos + iota
                    f0 = plsc.load_gather(rows_v, [srow, m0])
                    f1 = plsc.load_gather(rows_v, [srow, m0 + np.int32(1)])
                    acc0 = acc0 + wc * f0
                    acc1 = acc1 + wc * f1
                obase = (lax.shift_left(lax.shift_right_logical(
                    off, np.int32(7)), np.int32(10)) + (off & np.int32(127)))
                for f01, acc in ((0, acc0), (1, acc1)):
                    f = 2 * l + f01
                    sbase = obase + np.int32((f >> 3) * 2048 + (f & 7) * 128)
                    out_v[pl.ds(sbase, 16)] = acc

            _fori32(GROUPS, comb_body)

        bufs = ((idx_a, m_a, rows_a, sem_a), (idx_b, m_b, rows_b, sem_b))
        make_hash(0, idx_a, m_a)
        pltpu.async_copy(tab_hbm.at[idx_a], rows_a, sem_a)
        for l in range(N_LEVELS):
            idx_c, m_c, rows_c, sem_c = bufs[l & 1]
            if l + 1 < N_LEVELS:
                idx_n, m_n, rows_n, sem_n = bufs[(l + 1) & 1]
                make_hash(l + 1, idx_n, m_n)
                pltpu.async_copy(tab_hbm.at[idx_n], rows_n, sem_n)
            pltpu.make_async_copy(tab_hbm.at[idx_c], rows_c, sem_c).wait()
            make_comb(l, m_c, rows_c)
        ob = pl.multiple_of(
            lax.shift_left(lax.shift_right_logical(cbase, np.int32(7)),
                           np.int32(10)), 1024)
        for fh in range(3):
            pltpu.sync_copy(
                out_v.at[pl.ds(np.int32(fh * 2048), 2048)],
                out_hbm.at[pl.ds(ob + np.int32(fh * 4194304), 2048)])

    _fori32(N_CHUNKS, chunk_body)


@jax.jit
def _hash_grid(x_flat, tab_flat):
    mesh = plsc.VectorSubcoreMesh(core_axis_name="c", subcore_axis_name="s")
    params = pltpu.CompilerParams(needs_layout_passes=False,
                                  use_tc_tiling_on_sc=False)
    tab_il = pl.kernel(
        _il_body,
        out_type=jax.ShapeDtypeStruct((N_SUPER, SUPER_W), jnp.float32),
        mesh=mesh,
        compiler_params=params,
        scratch_types=[
            pltpu.VMEM((BLK_STEP * 256,), jnp.float32),
            pltpu.VMEM((BLK_STEP * 16, SUPER_W), jnp.float32),
        ],
    )(tab_flat)
    out = pl.kernel(
        _body,
        out_type=jax.ShapeDtypeStruct((N_PTS * N_OUT,), jnp.float32),
        mesh=mesh,
        compiler_params=params,
        scratch_types=[
            pltpu.VMEM((3 * CHUNK,), jnp.float32),
            pltpu.VMEM((N_IDX,), jnp.int32),
            pltpu.VMEM((N_IDX,), jnp.int32),
            pltpu.VMEM((N_IDX, SUPER_W), jnp.float32),
            pltpu.VMEM((N_IDX,), jnp.int32),
            pltpu.VMEM((N_IDX,), jnp.int32),
            pltpu.VMEM((N_IDX, SUPER_W), jnp.float32),
            pltpu.VMEM((CHUNK * N_OUT,), jnp.float32),
            pltpu.SemaphoreType.DMA,
            pltpu.SemaphoreType.DMA,
        ],
    )(x_flat, tab_il)
    return out.reshape(3, N_PTS // 128, 8, 128).transpose(
        1, 3, 0, 2).reshape(N_PTS, N_OUT)


def kernel(x, tables):
    x_flat = x.astype(jnp.float32).reshape(-1)
    tab_flat = tables.astype(jnp.float32).reshape(
        N_LEVELS, HASHMAP_SIZE // 128, 128, N_FEATURES).transpose(
        0, 1, 3, 2).reshape(-1)
    with _jax_src_config.enable_x64(False):
        return _hash_grid(x_flat, tab_flat)
